# Initial kernel scaffold; baseline (speedup 1.0000x reference)
import jax
import jax.numpy as jnp
from jax.experimental import pallas as pl


def kernel(x, adj, W1, b1, bn_gamma, bn_beta, bn_mean, bn_var, Wa, ba):
    raise NotImplementedError("write your pallas kernel here")


# speedup vs baseline: 1.9918x; 1.9918x over previous
import functools

import jax
import jax.numpy as jnp
from jax.experimental import pallas as pl

N = 4096
BM = 512


def _prep_kernel(x_ref, w1_ref, scale_ref, t_ref):
    t = jnp.dot(x_ref[...].astype(jnp.bfloat16), w1_ref[...].astype(jnp.bfloat16),
                preferred_element_type=jnp.float32)
    t_ref[...] = (t * scale_ref[...]).astype(jnp.bfloat16)


def _hidden_kernel(adj_ref, t_ref, c_ref, wa_ref, p_ref):
    h = jnp.dot(adj_ref[...].astype(jnp.bfloat16), t_ref[...],
                preferred_element_type=jnp.float32)
    h = jnp.maximum(h + c_ref[...], 0.0)
    p = jnp.dot(h.astype(jnp.bfloat16), wa_ref[...],
                preferred_element_type=jnp.float32)
    p_ref[...] = p.astype(jnp.bfloat16)


def _out_kernel(adj_ref, p_ref, ba_ref, o_ref):
    logits = jnp.dot(adj_ref[...].astype(jnp.bfloat16), p_ref[...],
                     preferred_element_type=jnp.float32)
    logits = logits + ba_ref[...]
    m = jnp.max(logits, axis=1, keepdims=True)
    s = logits - m
    o_ref[...] = s - jnp.log(jnp.sum(jnp.exp(s), axis=1, keepdims=True))


@functools.partial(jax.jit, static_argnames=())
def kernel(x, adj, W1, b1, bn_gamma, bn_beta, bn_mean, bn_var, Wa, ba):
    nfeat = x.shape[1]
    nhid = W1.shape[1]
    nheads, _, nclass = Wa.shape
    ncat = nheads * nclass

    scale = bn_gamma / jnp.sqrt(bn_var + 1e-5)
    c = ((b1 - bn_mean) * scale + bn_beta).reshape(1, nhid)
    scale = scale.reshape(1, nhid)
    wa_cat = jnp.transpose(Wa, (1, 0, 2)).reshape(nhid, ncat).astype(jnp.bfloat16)
    ba_cat = ba.reshape(1, ncat)

    t = pl.pallas_call(
        _prep_kernel,
        out_shape=jax.ShapeDtypeStruct((N, nhid), jnp.bfloat16),
    )(x, W1, scale)

    nb = N // BM
    p = pl.pallas_call(
        _hidden_kernel,
        grid=(nb,),
        in_specs=[
            pl.BlockSpec((BM, N), lambda i: (i, 0)),
            pl.BlockSpec((N, nhid), lambda i: (0, 0)),
            pl.BlockSpec((1, nhid), lambda i: (0, 0)),
            pl.BlockSpec((nhid, ncat), lambda i: (0, 0)),
        ],
        out_specs=pl.BlockSpec((BM, ncat), lambda i: (i, 0)),
        out_shape=jax.ShapeDtypeStruct((N, ncat), jnp.bfloat16),
    )(adj, t, c, wa_cat)

    out = pl.pallas_call(
        _out_kernel,
        grid=(nb,),
        in_specs=[
            pl.BlockSpec((BM, N), lambda i: (i, 0)),
            pl.BlockSpec((N, ncat), lambda i: (0, 0)),
            pl.BlockSpec((1, ncat), lambda i: (0, 0)),
        ],
        out_specs=pl.BlockSpec((BM, ncat), lambda i: (i, 0)),
        out_shape=jax.ShapeDtypeStruct((N, ncat), jnp.float32),
    )(adj, p, ba_cat)
    return out


# speedup vs baseline: 2.3923x; 1.2011x over previous
import jax
import jax.numpy as jnp
from jax.experimental import pallas as pl
from jax.experimental.pallas import tpu as pltpu

N = 4096
BM = 512
NB = N // BM


def _gcn_kernel(adj_ref, x_ref, w1_ref, scale_ref, c_ref, wa_ref, ba_ref,
                o_ref, adjv_ref, t_ref, p_ref):
    s = pl.program_id(0)
    i = pl.program_id(1)

    @pl.when(s == 0)
    def _stage0():
        @pl.when(i == 0)
        def _prep():
            t = jnp.dot(x_ref[...], w1_ref[...],
                        preferred_element_type=jnp.float32)
            t_ref[...] = (t * scale_ref[...]).astype(jnp.bfloat16)

        ablk = adj_ref[...].astype(jnp.bfloat16)
        adjv_ref[pl.ds(i * BM, BM), :] = ablk
        h = jnp.dot(ablk, t_ref[...], preferred_element_type=jnp.float32)
        h = jnp.maximum(h + c_ref[...], 0.0)
        p_ref[pl.ds(i * BM, BM), :] = jnp.dot(
            h.astype(jnp.bfloat16), wa_ref[...],
            preferred_element_type=jnp.float32).astype(jnp.bfloat16)

    @pl.when(s == 1)
    def _stage1():
        logits = jnp.dot(adjv_ref[pl.ds(i * BM, BM), :], p_ref[...],
                         preferred_element_type=jnp.float32)
        logits = logits + ba_ref[...]
        m = jnp.max(logits, axis=1, keepdims=True)
        z = logits - m
        o_ref[...] = z - jnp.log(jnp.sum(jnp.exp(z), axis=1, keepdims=True))


def kernel(x, adj, W1, b1, bn_gamma, bn_beta, bn_mean, bn_var, Wa, ba):
    nhid = W1.shape[1]
    nheads, _, nclass = Wa.shape
    ncat = nheads * nclass

    scale = bn_gamma / jnp.sqrt(bn_var + 1e-5)
    c = ((b1 - bn_mean) * scale + bn_beta).reshape(1, nhid)
    scale = scale.reshape(1, nhid)
    wa_cat = jnp.transpose(Wa, (1, 0, 2)).reshape(nhid, ncat).astype(jnp.bfloat16)
    ba_cat = ba.reshape(1, ncat)
    xb = x.astype(jnp.bfloat16)
    w1b = W1.astype(jnp.bfloat16)

    out = pl.pallas_call(
        _gcn_kernel,
        grid=(2, NB),
        in_specs=[
            pl.BlockSpec((BM, N), lambda s, i: (jnp.where(s == 0, i, NB - 1), 0)),
            pl.BlockSpec((N, x.shape[1]), lambda s, i: (0, 0)),
            pl.BlockSpec((x.shape[1], nhid), lambda s, i: (0, 0)),
            pl.BlockSpec((1, nhid), lambda s, i: (0, 0)),
            pl.BlockSpec((1, nhid), lambda s, i: (0, 0)),
            pl.BlockSpec((nhid, ncat), lambda s, i: (0, 0)),
            pl.BlockSpec((1, ncat), lambda s, i: (0, 0)),
        ],
        out_specs=pl.BlockSpec((BM, ncat),
                               lambda s, i: (jnp.where(s == 0, 0, i), 0)),
        out_shape=jax.ShapeDtypeStruct((N, ncat), jnp.float32),
        scratch_shapes=[
            pltpu.VMEM((N, N), jnp.bfloat16),
            pltpu.VMEM((N, nhid), jnp.bfloat16),
            pltpu.VMEM((N, ncat), jnp.bfloat16),
        ],
    )(adj, xb, w1b, scale, c, wa_cat, ba_cat)
    return out
